# baseline (speedup 1.0000x reference)
import functools

import jax
import jax.numpy as jnp
from jax import lax
from jax.experimental import pallas as pl
from jax.experimental.pallas import tpu as pltpu
from jax.experimental.pallas import tpu_sc as plsc

N = 2
T = 2048
D_MODEL = 2048
DIM = 2048
NUM_HEADS = 16
HEAD_DIM = DIM // NUM_HEADS
NUM_HASHES = 2
BUCKET_SIZE = 64
N_BUCKETS = T // BUCKET_SIZE
N_CHUNKS = NUM_HASHES * N_BUCKETS
SEQ_SORT = NUM_HASHES * T
NEG = -1e9

_INTERPRET = False


def _rot_matrix():
    rkey = jax.random.key(42)
    rots = []
    for h in range(NUM_HEADS):
        kh = jax.random.fold_in(rkey, h)
        r = jax.random.normal(kh, (1, HEAD_DIM, NUM_HASHES, N_BUCKETS // 2),
                              dtype=jnp.float32)
        rots.append(r[0])
    rot = jnp.stack(rots, axis=0)
    return rot.reshape(NUM_HEADS, HEAD_DIM, NUM_HASHES * (N_BUCKETS // 2))


def _proj_kernel(x_ref, wq_ref, bq_ref, wv_ref, bv_ref, rot_ref,
                 qh_ref, vh_ref, bk0_ref, bk1_ref):
    x = x_ref[0]
    q = jnp.dot(x, wq_ref[...], preferred_element_type=jnp.float32) + bq_ref[...]
    v = jnp.dot(x, wv_ref[...], preferred_element_type=jnp.float32) + bv_ref[...]
    bt = q.shape[0]
    for h in range(NUM_HEADS):
        qh = q[:, h * HEAD_DIM:(h + 1) * HEAD_DIM]
        qh_ref[h, 0] = qh
        vh_ref[h, 0] = v[:, h * HEAD_DIM:(h + 1) * HEAD_DIM]
        rh = rot_ref[h]
        hh = jnp.dot(qh, rh, preferred_element_type=jnp.float32)
        for hsh, out_ref in ((0, bk0_ref), (1, bk1_ref)):
            r = hh[:, hsh * (N_BUCKETS // 2):(hsh + 1) * (N_BUCKETS // 2)]
            full = jnp.concatenate([r, -r], axis=1)
            m = jnp.max(full, axis=1, keepdims=True)
            iota = lax.broadcasted_iota(jnp.int32, (bt, N_BUCKETS), 1)
            idx = jnp.min(jnp.where(full == m, iota, N_BUCKETS), axis=1,
                          keepdims=True)
            out_ref[0, :, h:h + 1] = idx + hsh * N_BUCKETS


def _projection(x, w_q, b_q, w_v, b_v, rot):
    bt = 256
    grid = (N, T // bt)
    kernel_fn = pl.pallas_call(
        _proj_kernel,
        grid=grid,
        in_specs=[
            pl.BlockSpec((1, bt, D_MODEL), lambda n, t: (n, t, 0)),
            pl.BlockSpec((D_MODEL, DIM), lambda n, t: (0, 0)),
            pl.BlockSpec((1, DIM), lambda n, t: (0, 0)),
            pl.BlockSpec((D_MODEL, DIM), lambda n, t: (0, 0)),
            pl.BlockSpec((1, DIM), lambda n, t: (0, 0)),
            pl.BlockSpec((NUM_HEADS, HEAD_DIM, N_BUCKETS), lambda n, t: (0, 0, 0)),
        ],
        out_specs=[
            pl.BlockSpec((NUM_HEADS, 1, bt, HEAD_DIM), lambda n, t: (0, n, t, 0)),
            pl.BlockSpec((NUM_HEADS, 1, bt, HEAD_DIM), lambda n, t: (0, n, t, 0)),
            pl.BlockSpec((1, bt, NUM_HEADS), lambda n, t: (n, t, 0)),
            pl.BlockSpec((1, bt, NUM_HEADS), lambda n, t: (n, t, 0)),
        ],
        out_shape=[
            jax.ShapeDtypeStruct((NUM_HEADS, N, T, HEAD_DIM), jnp.float32),
            jax.ShapeDtypeStruct((NUM_HEADS, N, T, HEAD_DIM), jnp.float32),
            jax.ShapeDtypeStruct((N, T, NUM_HEADS), jnp.int32),
            jax.ShapeDtypeStruct((N, T, NUM_HEADS), jnp.int32),
        ],
        interpret=_INTERPRET,
    )
    return kernel_fn(x, w_q, b_q.reshape(1, DIM), w_v, b_v.reshape(1, DIM), rot)


CHW = 128
N_GCHUNK = SEQ_SORT // CHW


def _sc_sort_gather(bkt, qh_flat, vh_flat):
    G = N * NUM_HEADS
    mesh = plsc.VectorSubcoreMesh(core_axis_name="c", subcore_axis_name="s")

    @functools.partial(
        pl.kernel, mesh=mesh,
        compiler_params=pltpu.CompilerParams(needs_layout_passes=False),
        out_type=[
            jax.ShapeDtypeStruct((G, SEQ_SORT, HEAD_DIM), jnp.float32),
            jax.ShapeDtypeStruct((G, SEQ_SORT, HEAD_DIM), jnp.float32),
            jax.ShapeDtypeStruct((G, SEQ_SORT), jnp.int32),
            jax.ShapeDtypeStruct((G, SEQ_SORT), jnp.int32),
            jax.ShapeDtypeStruct((G, SEQ_SORT), jnp.int32),
        ],
        scratch_types=[
            pltpu.VMEM((SEQ_SORT,), jnp.int32),
            pltpu.VMEM((SEQ_SORT + 16,), jnp.int32),
            pltpu.VMEM((SEQ_SORT,), jnp.int32),
            pltpu.VMEM((SEQ_SORT,), jnp.int32),
            pltpu.VMEM((SEQ_SORT,), jnp.int32),
            pltpu.VMEM((CHW,), jnp.int32),
            pltpu.VMEM((CHW,), jnp.int32),
            pltpu.VMEM((SEQ_SORT,), jnp.int32),
            pltpu.VMEM((CHW, HEAD_DIM), jnp.float32),
            pltpu.VMEM((CHW, HEAD_DIM), jnp.float32),
            pltpu.VMEM((CHW, HEAD_DIM), jnp.float32),
            pltpu.VMEM((CHW, HEAD_DIM), jnp.float32),
            pltpu.SemaphoreType.DMA,
            pltpu.SemaphoreType.DMA,
            pltpu.SemaphoreType.DMA,
            pltpu.SemaphoreType.DMA,
        ],
    )
    def k(bkt_hbm, qh_hbm, vh_hbm,
          sqk_hbm, sv_hbm, st_hbm, sbkt_hbm, undo_hbm,
          bkt_v, ids_v, st_v, sbkt_v, qidx_v, idx_a, idx_b,
          undo_v, qr_a, qr_b, vr_a, vr_b, sq_a, sq_b, sv_a, sv_b):
        cid = lax.axis_index("c")
        sid = lax.axis_index("s")
        w = sid * 2 + cid
        n = w // NUM_HEADS
        h = lax.rem(w, NUM_HEADS)
        base = (h * N + n) * T

        pltpu.sync_copy(bkt_hbm.at[w], bkt_v)
        lane = lax.iota(jnp.int32, 16)

        def compact(vlo, glo, ghi, off0):
            def outer(vi, off):
                v = vlo + vi

                def inner(g, off2):
                    vec = bkt_v[pl.ds(g * 16, 16)]
                    mask = vec == v
                    idxs = g * 16 + lane
                    csum = plsc.cumsum(mask.astype(jnp.int32))
                    plsc.store_scatter(ids_v, [off2 + csum - 1], idxs,
                                       mask=mask)
                    return off2 + csum[15]
                return lax.fori_loop(glo, ghi, inner, off)
            return lax.fori_loop(0, N_BUCKETS, outer, off0)

        off = compact(0, 0, T // 16, 0)
        compact(N_BUCKETS, T // 16, SEQ_SORT // 16, off)

        def derive(g, _):
            idv = ids_v[pl.ds(g * 16, 16)]
            sb = plsc.load_gather(bkt_v, [idv])
            t = lax.rem(idv, T)
            st_v[pl.ds(g * 16, 16)] = t
            sbkt_v[pl.ds(g * 16, 16)] = sb
            qidx_v[pl.ds(g * 16, 16)] = base + t
            plsc.store_scatter(undo_v, [idv], g * 16 + lane)
            return 0
        lax.fori_loop(0, SEQ_SORT // 16, derive, 0)

        pltpu.sync_copy(st_v, st_hbm.at[w])
        pltpu.sync_copy(sbkt_v, sbkt_hbm.at[w])
        pltpu.sync_copy(undo_v, undo_hbm.at[w])

        ibufs = (idx_a, idx_b)
        qrows = (qr_a, qr_b)
        vrows = (vr_a, vr_b)
        qsems = (sq_a, sq_b)
        vsems = (sv_a, sv_b)

        def fill_idx(c, dst):
            for i in range(CHW // 16):
                dst[pl.ds(i * 16, 16)] = qidx_v[pl.ds(c * CHW + i * 16, 16)]

        fill_idx(0, ibufs[0])
        pltpu.async_copy(qh_hbm.at[ibufs[0]], qrows[0], qsems[0])
        pltpu.async_copy(vh_hbm.at[ibufs[0]], vrows[0], vsems[0])

        def gather(i, _):
            for b in (0, 1):
                c = 2 * i + b
                nb = 1 - b

                @pl.when(c + 1 < N_GCHUNK)
                def _():
                    fill_idx(c + 1, ibufs[nb])
                    pltpu.async_copy(qh_hbm.at[ibufs[nb]], qrows[nb],
                                     qsems[nb])
                    pltpu.async_copy(vh_hbm.at[ibufs[nb]], vrows[nb],
                                     vsems[nb])
                pltpu.make_async_copy(qh_hbm.at[ibufs[b]], qrows[b],
                                      qsems[b]).wait()
                pltpu.sync_copy(qrows[b], sqk_hbm.at[w, pl.ds(c * CHW, CHW), :])
                pltpu.make_async_copy(vh_hbm.at[ibufs[b]], vrows[b],
                                      vsems[b]).wait()
                pltpu.sync_copy(vrows[b], sv_hbm.at[w, pl.ds(c * CHW, CHW), :])
            return 0
        lax.fori_loop(0, N_GCHUNK // 2, gather, 0)

    return k(bkt, qh_flat, vh_flat)


TCH = 128
N_TCHUNK = T // TCH


def _sc_unsort_combine(so_flat, lse, undo):
    G = N * NUM_HEADS
    mesh = plsc.VectorSubcoreMesh(core_axis_name="c", subcore_axis_name="s")

    @functools.partial(
        pl.kernel, mesh=mesh,
        compiler_params=pltpu.CompilerParams(needs_layout_passes=False),
        out_type=[
            jax.ShapeDtypeStruct((N * T, DIM), jnp.float32),
        ],
        scratch_types=[
            pltpu.VMEM((SEQ_SORT,), jnp.float32),
            pltpu.VMEM((SEQ_SORT,), jnp.int32),
            pltpu.VMEM((T,), jnp.float32),
            pltpu.VMEM((T,), jnp.float32),
            pltpu.VMEM((TCH,), jnp.int32),
            pltpu.VMEM((TCH,), jnp.int32),
            pltpu.VMEM((TCH,), jnp.int32),
            pltpu.VMEM((TCH,), jnp.int32),
            pltpu.VMEM((TCH, HEAD_DIM), jnp.float32),
            pltpu.VMEM((TCH, HEAD_DIM), jnp.float32),
            pltpu.VMEM((TCH, HEAD_DIM), jnp.float32),
            pltpu.VMEM((TCH, HEAD_DIM), jnp.float32),
            pltpu.SemaphoreType.DMA,
            pltpu.SemaphoreType.DMA,
            pltpu.SemaphoreType.DMA,
            pltpu.SemaphoreType.DMA,
        ],
    )
    def k(so_hbm, lse_hbm, undo_hbm, o_hbm,
          lse_v, undo_v, w0_v, w1_v, i0_a, i0_b, i1_a, i1_b,
          r0_a, r0_b, r1_a, r1_b, s0_a, s0_b, s1_a, s1_b):
        cid = lax.axis_index("c")
        sid = lax.axis_index("s")
        w = sid * 2 + cid
        n = w // NUM_HEADS
        h = lax.rem(w, NUM_HEADS)
        srow = w * SEQ_SORT

        pltpu.sync_copy(lse_hbm.at[w], lse_v)
        pltpu.sync_copy(undo_hbm.at[w], undo_v)

        def weights(g, _):
            p0 = undo_v[pl.ds(g * 16, 16)]
            p1 = undo_v[pl.ds(T + g * 16, 16)]
            l0 = plsc.load_gather(lse_v, [p0])
            l1 = plsc.load_gather(lse_v, [p1])
            m = jnp.maximum(l0, l1)
            e0 = jnp.exp(l0 - m)
            e1 = jnp.exp(l1 - m)
            s = e0 + e1
            w0_v[pl.ds(g * 16, 16)] = e0 / s
            w1_v[pl.ds(g * 16, 16)] = e1 / s
            return 0
        lax.fori_loop(0, T // 16, weights, 0)

        i0 = (i0_a, i0_b)
        i1 = (i1_a, i1_b)
        r0 = (r0_a, r0_b)
        r1 = (r1_a, r1_b)
        s0 = (s0_a, s0_b)
        s1 = (s1_a, s1_b)

        def fill_idx(c, d0, d1):
            for i in range(TCH // 16):
                d0[pl.ds(i * 16, 16)] = srow + undo_v[
                    pl.ds(c * TCH + i * 16, 16)]
                d1[pl.ds(i * 16, 16)] = srow + undo_v[
                    pl.ds(T + c * TCH + i * 16, 16)]

        fill_idx(0, i0[0], i1[0])
        pltpu.async_copy(so_hbm.at[i0[0]], r0[0], s0[0])
        pltpu.async_copy(so_hbm.at[i1[0]], r1[0], s1[0])

        def body(i, _):
            for b in (0, 1):
                c = 2 * i + b
                nb = 1 - b

                @pl.when(c + 1 < N_TCHUNK)
                def _():
                    fill_idx(c + 1, i0[nb], i1[nb])
                    pltpu.async_copy(so_hbm.at[i0[nb]], r0[nb], s0[nb])
                    pltpu.async_copy(so_hbm.at[i1[nb]], r1[nb], s1[nb])
                pltpu.make_async_copy(so_hbm.at[i0[b]], r0[b], s0[b]).wait()
                pltpu.make_async_copy(so_hbm.at[i1[b]], r1[b], s1[b]).wait()

                def comb(rg, _):
                    wv0 = w0_v[pl.ds(c * TCH + rg * 16, 16)]
                    wv1 = w1_v[pl.ds(c * TCH + rg * 16, 16)]
                    for rr in range(16):
                        a = wv0[rr]
                        bb = wv1[rr]
                        r = rg * 16 + rr
                        for u in range(HEAD_DIM // 16):
                            r0[b][r, pl.ds(u * 16, 16)] = (
                                a * r0[b][r, pl.ds(u * 16, 16)]
                                + bb * r1[b][r, pl.ds(u * 16, 16)])
                    return 0
                lax.fori_loop(0, TCH // 16, comb, 0)
                pltpu.sync_copy(
                    r0[b],
                    o_hbm.at[pl.ds(n * T + c * TCH, TCH),
                             pl.ds(h * HEAD_DIM, HEAD_DIM)])
            return 0
        lax.fori_loop(0, N_TCHUNK // 2, body, 0)

    return k(so_flat, lse, undo)


CPG = 8
GQ = CPG * BUCKET_SIZE
GK = GQ + BUCKET_SIZE
N_GROUPS = N_CHUNKS // CPG


def _attn_kernel(seq_ref, sqk_ref, sv_ref, tcol_ref, tkvh_ref, bcol_ref,
                 bkvh_ref, so_ref, lse_ref):
    seq_len = seq_ref[0, 0]
    scale = float(HEAD_DIM) ** -0.5
    cs = BUCKET_SIZE

    rc = lax.broadcasted_iota(jnp.int32, (GQ, GK), 0) // cs
    jc = lax.broadcasted_iota(jnp.int32, (GQ, GK), 1) // cs
    in_window = (jc == rc) | (jc == rc + 1)

    def body(g, _):
        lo = g * GQ
        po = (lax.rem(g * CPG + N_CHUNKS - 1, N_CHUNKS)) * cs
        q = sqk_ref[0, pl.ds(lo, GQ), :]
        kp = sqk_ref[0, pl.ds(po, cs), :]
        k = jnp.concatenate([kp, q], axis=0)
        vp = sv_ref[0, pl.ds(po, cs), :]
        v = jnp.concatenate([vp, sv_ref[0, pl.ds(lo, GQ), :]], axis=0)
        knorm = k / (jnp.sqrt(jnp.sum(k * k, axis=1, keepdims=True)) + 1e-6)
        dots = lax.dot_general(q, knorm, (((1,), (1,)), ((), ())),
                               preferred_element_type=jnp.float32) * scale
        tq = tcol_ref[0, pl.ds(lo, GQ), :]
        tkv = tkvh_ref[0, g]
        qb = bcol_ref[0, pl.ds(lo, GQ), :]
        kvb = bkvh_ref[0, g]
        bad = (qb != kvb) | (tkv > tq) | (tq >= seq_len) | (tkv >= seq_len)
        dots = jnp.where(bad, NEG, dots)
        dots = jnp.where(tq == tkv, -1e-5, dots)
        dots = jnp.where(in_window, dots, NEG)
        m = jnp.max(dots, axis=1, keepdims=True)
        e = jnp.exp(dots - m)
        s = jnp.sum(e, axis=1, keepdims=True)
        lse = jnp.log(s) + m
        p = e / s
        bo = lax.dot_general(p, v, (((1,), (0,)), ((), ())),
                             preferred_element_type=jnp.float32)
        so_ref[0, pl.ds(lo, GQ), :] = bo
        lse_ref[0, pl.ds(lo, GQ), :] = lse
        return 0

    lax.fori_loop(0, N_GROUPS, body, 0)


def _halo(x):
    xc = x.reshape(-1, N_GROUPS, GQ)
    xp = jnp.roll(x.reshape(-1, N_CHUNKS, BUCKET_SIZE), 1, axis=1)[:, 0::CPG]
    return jnp.concatenate([xp, xc], axis=2)[:, :, None, :]


def _attention(sqk, sv, st, sbkt, seq_len):
    g = N * NUM_HEADS
    tcol = st[..., None]
    bcol = sbkt[..., None]
    tkvh = _halo(st)
    bkvh = _halo(sbkt)
    seq = jnp.asarray(seq_len, dtype=jnp.int32).reshape(1, 1)
    kernel_fn = pl.pallas_call(
        _attn_kernel,
        grid=(g,),
        in_specs=[
            pl.BlockSpec(memory_space=pltpu.SMEM),
            pl.BlockSpec((1, SEQ_SORT, HEAD_DIM), lambda i: (i, 0, 0)),
            pl.BlockSpec((1, SEQ_SORT, HEAD_DIM), lambda i: (i, 0, 0)),
            pl.BlockSpec((1, SEQ_SORT, 1), lambda i: (i, 0, 0)),
            pl.BlockSpec((1, N_GROUPS, 1, GK), lambda i: (i, 0, 0, 0)),
            pl.BlockSpec((1, SEQ_SORT, 1), lambda i: (i, 0, 0)),
            pl.BlockSpec((1, N_GROUPS, 1, GK), lambda i: (i, 0, 0, 0)),
        ],
        out_specs=[
            pl.BlockSpec((1, SEQ_SORT, HEAD_DIM), lambda i: (i, 0, 0)),
            pl.BlockSpec((1, SEQ_SORT, 1), lambda i: (i, 0, 0)),
        ],
        out_shape=[
            jax.ShapeDtypeStruct((g, SEQ_SORT, HEAD_DIM), jnp.float32),
            jax.ShapeDtypeStruct((g, SEQ_SORT, 1), jnp.float32),
        ],
        interpret=_INTERPRET,
    )
    return kernel_fn(seq, sqk, sv, tcol, tkvh, bcol, bkvh)


def _ln_kernel(x_ref, g_ref, b_ref, out_ref):
    x = x_ref[0]
    mean = jnp.mean(x, axis=1, keepdims=True)
    var = jnp.mean((x - mean) ** 2, axis=1, keepdims=True)
    out_ref[0] = (x - mean) / jnp.sqrt(var + 1e-3) * g_ref[...] + b_ref[...]


def _layernorm(x, gamma, beta):
    bt = 256
    kernel_fn = pl.pallas_call(
        _ln_kernel,
        grid=(N, T // bt),
        in_specs=[
            pl.BlockSpec((1, bt, DIM), lambda n, t: (n, t, 0)),
            pl.BlockSpec((1, DIM), lambda n, t: (0, 0)),
            pl.BlockSpec((1, DIM), lambda n, t: (0, 0)),
        ],
        out_specs=pl.BlockSpec((1, bt, DIM), lambda n, t: (n, t, 0)),
        out_shape=jax.ShapeDtypeStruct((N, T, DIM), jnp.float32),
        interpret=_INTERPRET,
    )
    return kernel_fn(x, gamma.reshape(1, DIM), beta.reshape(1, DIM))


def kernel(inputs, W_Q, b_Q, W_V, b_V, gamma, beta, seq_len):
    rot = _rot_matrix()
    qh, vh, bk0, bk1 = _projection(inputs, W_Q, b_Q, W_V, b_V, rot)

    bkt = jnp.stack([bk0, bk1], axis=1)
    bkt = bkt.transpose(0, 3, 1, 2)
    bkt = bkt.reshape(N * NUM_HEADS, SEQ_SORT)

    qh_flat = qh.reshape(NUM_HEADS * N * T, HEAD_DIM)
    vh_flat = vh.reshape(NUM_HEADS * N * T, HEAD_DIM)
    sqk, sv, st, sbkt, undo = _sc_sort_gather(bkt, qh_flat, vh_flat)

    so, lse = _attention(sqk, sv, st, sbkt, seq_len)

    o_comb = _sc_unsort_combine(
        so.reshape(N * NUM_HEADS * SEQ_SORT, HEAD_DIM), lse[..., 0], undo)
    if isinstance(o_comb, (list, tuple)):
        o_comb = o_comb[0]

    return _layernorm(o_comb.reshape(N, T, DIM), gamma, beta)


# speedup vs baseline: 1.0337x; 1.0337x over previous
import functools

import jax
import jax.numpy as jnp
from jax import lax
from jax.experimental import pallas as pl
from jax.experimental.pallas import tpu as pltpu
from jax.experimental.pallas import tpu_sc as plsc

N = 2
T = 2048
D_MODEL = 2048
DIM = 2048
NUM_HEADS = 16
HEAD_DIM = DIM // NUM_HEADS
NUM_HASHES = 2
BUCKET_SIZE = 64
N_BUCKETS = T // BUCKET_SIZE
N_CHUNKS = NUM_HASHES * N_BUCKETS
SEQ_SORT = NUM_HASHES * T
NEG = -1e9

_INTERPRET = False


def _rot_matrix():
    rkey = jax.random.key(42)
    rots = []
    for h in range(NUM_HEADS):
        kh = jax.random.fold_in(rkey, h)
        r = jax.random.normal(kh, (1, HEAD_DIM, NUM_HASHES, N_BUCKETS // 2),
                              dtype=jnp.float32)
        rots.append(r[0])
    rot = jnp.stack(rots, axis=0)
    return rot.reshape(NUM_HEADS, HEAD_DIM, NUM_HASHES * (N_BUCKETS // 2))


def _proj_kernel(x_ref, wq_ref, bq_ref, wv_ref, bv_ref, rot_ref,
                 qh_ref, vh_ref, bk0_ref, bk1_ref):
    x = x_ref[0]
    q = jnp.dot(x, wq_ref[...], preferred_element_type=jnp.float32) + bq_ref[...]
    v = jnp.dot(x, wv_ref[...], preferred_element_type=jnp.float32) + bv_ref[...]
    bt = q.shape[0]
    for h in range(NUM_HEADS):
        qh = q[:, h * HEAD_DIM:(h + 1) * HEAD_DIM]
        qh_ref[h, 0] = qh
        vh_ref[h, 0] = v[:, h * HEAD_DIM:(h + 1) * HEAD_DIM]
        rh = rot_ref[h]
        hh = jnp.dot(qh, rh, preferred_element_type=jnp.float32)
        for hsh, out_ref in ((0, bk0_ref), (1, bk1_ref)):
            r = hh[:, hsh * (N_BUCKETS // 2):(hsh + 1) * (N_BUCKETS // 2)]
            full = jnp.concatenate([r, -r], axis=1)
            m = jnp.max(full, axis=1, keepdims=True)
            iota = lax.broadcasted_iota(jnp.int32, (bt, N_BUCKETS), 1)
            idx = jnp.min(jnp.where(full == m, iota, N_BUCKETS), axis=1,
                          keepdims=True)
            out_ref[0, :, h:h + 1] = idx + hsh * N_BUCKETS


def _projection(x, w_q, b_q, w_v, b_v, rot):
    bt = 256
    grid = (N, T // bt)
    kernel_fn = pl.pallas_call(
        _proj_kernel,
        grid=grid,
        in_specs=[
            pl.BlockSpec((1, bt, D_MODEL), lambda n, t: (n, t, 0)),
            pl.BlockSpec((D_MODEL, DIM), lambda n, t: (0, 0)),
            pl.BlockSpec((1, DIM), lambda n, t: (0, 0)),
            pl.BlockSpec((D_MODEL, DIM), lambda n, t: (0, 0)),
            pl.BlockSpec((1, DIM), lambda n, t: (0, 0)),
            pl.BlockSpec((NUM_HEADS, HEAD_DIM, N_BUCKETS), lambda n, t: (0, 0, 0)),
        ],
        out_specs=[
            pl.BlockSpec((NUM_HEADS, 1, bt, HEAD_DIM), lambda n, t: (0, n, t, 0)),
            pl.BlockSpec((NUM_HEADS, 1, bt, HEAD_DIM), lambda n, t: (0, n, t, 0)),
            pl.BlockSpec((1, bt, NUM_HEADS), lambda n, t: (n, t, 0)),
            pl.BlockSpec((1, bt, NUM_HEADS), lambda n, t: (n, t, 0)),
        ],
        out_shape=[
            jax.ShapeDtypeStruct((NUM_HEADS, N, T, HEAD_DIM), jnp.float32),
            jax.ShapeDtypeStruct((NUM_HEADS, N, T, HEAD_DIM), jnp.float32),
            jax.ShapeDtypeStruct((N, T, NUM_HEADS), jnp.int32),
            jax.ShapeDtypeStruct((N, T, NUM_HEADS), jnp.int32),
        ],
        interpret=_INTERPRET,
    )
    return kernel_fn(x, w_q, b_q.reshape(1, DIM), w_v, b_v.reshape(1, DIM), rot)


CHW = 128
N_GCHUNK = SEQ_SORT // CHW


def _sc_sort_gather(bkt, qh_flat, vh_flat):
    G = N * NUM_HEADS
    mesh = plsc.VectorSubcoreMesh(core_axis_name="c", subcore_axis_name="s")

    @functools.partial(
        pl.kernel, mesh=mesh,
        compiler_params=pltpu.CompilerParams(needs_layout_passes=False),
        out_type=[
            jax.ShapeDtypeStruct((G, SEQ_SORT, HEAD_DIM), jnp.float32),
            jax.ShapeDtypeStruct((G, SEQ_SORT, HEAD_DIM), jnp.float32),
            jax.ShapeDtypeStruct((G, SEQ_SORT), jnp.int32),
            jax.ShapeDtypeStruct((G, SEQ_SORT), jnp.int32),
            jax.ShapeDtypeStruct((G, SEQ_SORT), jnp.int32),
        ],
        scratch_types=[
            pltpu.VMEM((SEQ_SORT,), jnp.int32),
            pltpu.VMEM((SEQ_SORT + 16,), jnp.int32),
            pltpu.VMEM((SEQ_SORT,), jnp.int32),
            pltpu.VMEM((SEQ_SORT,), jnp.int32),
            pltpu.VMEM((SEQ_SORT,), jnp.int32),
            pltpu.VMEM((CHW,), jnp.int32),
            pltpu.VMEM((CHW,), jnp.int32),
            pltpu.VMEM((SEQ_SORT,), jnp.int32),
            pltpu.VMEM((CHW, HEAD_DIM), jnp.float32),
            pltpu.VMEM((CHW, HEAD_DIM), jnp.float32),
            pltpu.VMEM((CHW, HEAD_DIM), jnp.float32),
            pltpu.VMEM((CHW, HEAD_DIM), jnp.float32),
            pltpu.SemaphoreType.DMA,
            pltpu.SemaphoreType.DMA,
            pltpu.SemaphoreType.DMA,
            pltpu.SemaphoreType.DMA,
        ],
    )
    def k(bkt_hbm, qh_hbm, vh_hbm,
          sqk_hbm, sv_hbm, st_hbm, sbkt_hbm, undo_hbm,
          bkt_v, ids_v, st_v, sbkt_v, qidx_v, idx_a, idx_b,
          undo_v, qr_a, qr_b, vr_a, vr_b, sq_a, sq_b, sv_a, sv_b):
        cid = lax.axis_index("c")
        sid = lax.axis_index("s")
        w = sid * 2 + cid
        n = w // NUM_HEADS
        h = lax.rem(w, NUM_HEADS)
        base = (h * N + n) * T

        pltpu.sync_copy(bkt_hbm.at[w], bkt_v)
        lane = lax.iota(jnp.int32, 16)

        def compact(vlo, glo, ghi, off0):
            def outer(vi, off):
                v = vlo + vi

                def inner(g, off2):
                    vec = bkt_v[pl.ds(g * 16, 16)]
                    mask = vec == v
                    idxs = g * 16 + lane
                    csum = plsc.cumsum(mask.astype(jnp.int32))
                    plsc.store_scatter(ids_v, [off2 + csum - 1], idxs,
                                       mask=mask)
                    return off2 + csum[15]
                return lax.fori_loop(glo, ghi, inner, off)
            return lax.fori_loop(0, N_BUCKETS, outer, off0)

        off = compact(0, 0, T // 16, 0)
        compact(N_BUCKETS, T // 16, SEQ_SORT // 16, off)

        def derive(g, _):
            idv = ids_v[pl.ds(g * 16, 16)]
            sb = plsc.load_gather(bkt_v, [idv])
            t = lax.rem(idv, T)
            st_v[pl.ds(g * 16, 16)] = t
            sbkt_v[pl.ds(g * 16, 16)] = sb
            qidx_v[pl.ds(g * 16, 16)] = base + t
            plsc.store_scatter(undo_v, [idv], g * 16 + lane)
            return 0
        lax.fori_loop(0, SEQ_SORT // 16, derive, 0)

        pltpu.sync_copy(st_v, st_hbm.at[w])
        pltpu.sync_copy(sbkt_v, sbkt_hbm.at[w])
        pltpu.sync_copy(undo_v, undo_hbm.at[w])

        ibufs = (idx_a, idx_b)
        qrows = (qr_a, qr_b)
        vrows = (vr_a, vr_b)
        qsems = (sq_a, sq_b)
        vsems = (sv_a, sv_b)

        def fill_idx(c, dst):
            for i in range(CHW // 16):
                dst[pl.ds(i * 16, 16)] = qidx_v[pl.ds(c * CHW + i * 16, 16)]

        fill_idx(0, ibufs[0])
        pltpu.async_copy(qh_hbm.at[ibufs[0]], qrows[0], qsems[0])
        pltpu.async_copy(vh_hbm.at[ibufs[0]], vrows[0], vsems[0])

        def gather(i, _):
            for b in (0, 1):
                c = 2 * i + b
                nb = 1 - b

                @pl.when(c + 1 < N_GCHUNK)
                def _():
                    fill_idx(c + 1, ibufs[nb])
                    pltpu.async_copy(qh_hbm.at[ibufs[nb]], qrows[nb],
                                     qsems[nb])
                    pltpu.async_copy(vh_hbm.at[ibufs[nb]], vrows[nb],
                                     vsems[nb])
                pltpu.make_async_copy(qh_hbm.at[ibufs[b]], qrows[b],
                                      qsems[b]).wait()
                pltpu.sync_copy(qrows[b], sqk_hbm.at[w, pl.ds(c * CHW, CHW), :])
                pltpu.make_async_copy(vh_hbm.at[ibufs[b]], vrows[b],
                                      vsems[b]).wait()
                pltpu.sync_copy(vrows[b], sv_hbm.at[w, pl.ds(c * CHW, CHW), :])
            return 0
        lax.fori_loop(0, N_GCHUNK // 2, gather, 0)

    return k(bkt, qh_flat, vh_flat)


TCH = 128
N_TCHUNK = T // TCH


def _sc_unsort_combine(so_flat, lse, undo):
    G = N * NUM_HEADS
    mesh = plsc.VectorSubcoreMesh(core_axis_name="c", subcore_axis_name="s")

    @functools.partial(
        pl.kernel, mesh=mesh,
        compiler_params=pltpu.CompilerParams(needs_layout_passes=False),
        out_type=[
            jax.ShapeDtypeStruct((N * T, DIM), jnp.float32),
        ],
        scratch_types=[
            pltpu.VMEM((SEQ_SORT,), jnp.float32),
            pltpu.VMEM((SEQ_SORT,), jnp.int32),
            pltpu.VMEM((T,), jnp.float32),
            pltpu.VMEM((T,), jnp.float32),
            pltpu.VMEM((TCH,), jnp.int32),
            pltpu.VMEM((TCH,), jnp.int32),
            pltpu.VMEM((TCH,), jnp.int32),
            pltpu.VMEM((TCH,), jnp.int32),
            pltpu.VMEM((TCH, HEAD_DIM), jnp.float32),
            pltpu.VMEM((TCH, HEAD_DIM), jnp.float32),
            pltpu.VMEM((TCH, HEAD_DIM), jnp.float32),
            pltpu.VMEM((TCH, HEAD_DIM), jnp.float32),
            pltpu.SemaphoreType.DMA,
            pltpu.SemaphoreType.DMA,
            pltpu.SemaphoreType.DMA,
            pltpu.SemaphoreType.DMA,
        ],
    )
    def k(so_hbm, lse_hbm, undo_hbm, o_hbm,
          lse_v, undo_v, w0_v, w1_v, i0_a, i0_b, i1_a, i1_b,
          r0_a, r0_b, r1_a, r1_b, s0_a, s0_b, s1_a, s1_b):
        cid = lax.axis_index("c")
        sid = lax.axis_index("s")
        w = sid * 2 + cid
        n = w // NUM_HEADS
        h = lax.rem(w, NUM_HEADS)
        srow = w * SEQ_SORT

        pltpu.sync_copy(lse_hbm.at[w], lse_v)
        pltpu.sync_copy(undo_hbm.at[w], undo_v)

        def weights(g, _):
            p0 = undo_v[pl.ds(g * 16, 16)]
            p1 = undo_v[pl.ds(T + g * 16, 16)]
            l0 = plsc.load_gather(lse_v, [p0])
            l1 = plsc.load_gather(lse_v, [p1])
            m = jnp.maximum(l0, l1)
            e0 = jnp.exp(l0 - m)
            e1 = jnp.exp(l1 - m)
            s = e0 + e1
            w0_v[pl.ds(g * 16, 16)] = e0 / s
            w1_v[pl.ds(g * 16, 16)] = e1 / s
            return 0
        lax.fori_loop(0, T // 16, weights, 0)

        i0 = (i0_a, i0_b)
        i1 = (i1_a, i1_b)
        r0 = (r0_a, r0_b)
        r1 = (r1_a, r1_b)
        s0 = (s0_a, s0_b)
        s1 = (s1_a, s1_b)

        def fill_idx(c, d0, d1):
            for i in range(TCH // 16):
                d0[pl.ds(i * 16, 16)] = srow + undo_v[
                    pl.ds(c * TCH + i * 16, 16)]
                d1[pl.ds(i * 16, 16)] = srow + undo_v[
                    pl.ds(T + c * TCH + i * 16, 16)]

        fill_idx(0, i0[0], i1[0])
        pltpu.async_copy(so_hbm.at[i0[0]], r0[0], s0[0])
        pltpu.async_copy(so_hbm.at[i1[0]], r1[0], s1[0])

        def body(i, _):
            for b in (0, 1):
                c = 2 * i + b
                nb = 1 - b

                @pl.when(c + 1 < N_TCHUNK)
                def _():
                    fill_idx(c + 1, i0[nb], i1[nb])
                    pltpu.async_copy(so_hbm.at[i0[nb]], r0[nb], s0[nb])
                    pltpu.async_copy(so_hbm.at[i1[nb]], r1[nb], s1[nb])
                pltpu.make_async_copy(so_hbm.at[i0[b]], r0[b], s0[b]).wait()
                pltpu.make_async_copy(so_hbm.at[i1[b]], r1[b], s1[b]).wait()

                def comb(rg, _):
                    wv0 = w0_v[pl.ds(c * TCH + rg * 16, 16)]
                    wv1 = w1_v[pl.ds(c * TCH + rg * 16, 16)]
                    for rr in range(16):
                        a = wv0[rr]
                        bb = wv1[rr]
                        r = rg * 16 + rr
                        for u in range(HEAD_DIM // 16):
                            r0[b][r, pl.ds(u * 16, 16)] = (
                                a * r0[b][r, pl.ds(u * 16, 16)]
                                + bb * r1[b][r, pl.ds(u * 16, 16)])
                    return 0
                lax.fori_loop(0, TCH // 16, comb, 0)
                pltpu.sync_copy(
                    r0[b],
                    o_hbm.at[pl.ds(n * T + c * TCH, TCH),
                             pl.ds(h * HEAD_DIM, HEAD_DIM)])
            return 0
        lax.fori_loop(0, N_TCHUNK // 2, body, 0)

    return k(so_flat, lse, undo)


CPG = 8
GQ = CPG * BUCKET_SIZE
GK = GQ + BUCKET_SIZE
N_GROUPS = N_CHUNKS // CPG


def _attn_kernel(seq_ref, sqk_ref, sv_ref, tcol_ref, tkvh_ref, bcol_ref,
                 bkvh_ref, so_ref, lse_ref):
    seq_len = seq_ref[0, 0]
    scale = float(HEAD_DIM) ** -0.5
    cs = BUCKET_SIZE

    rc = lax.broadcasted_iota(jnp.int32, (GQ, GK), 0) // cs
    jc = lax.broadcasted_iota(jnp.int32, (GQ, GK), 1) // cs
    in_window = (jc == rc) | (jc == rc + 1)

    def body(g, _):
        lo = g * GQ
        po = (lax.rem(g * CPG + N_CHUNKS - 1, N_CHUNKS)) * cs
        q = sqk_ref[0, pl.ds(lo, GQ), :]
        kp = sqk_ref[0, pl.ds(po, cs), :]
        k = jnp.concatenate([kp, q], axis=0)
        vp = sv_ref[0, pl.ds(po, cs), :]
        v = jnp.concatenate([vp, sv_ref[0, pl.ds(lo, GQ), :]], axis=0)
        knorm = k / (jnp.sqrt(jnp.sum(k * k, axis=1, keepdims=True)) + 1e-6)
        dots = lax.dot_general(q * scale, knorm, (((1,), (1,)), ((), ())),
                               preferred_element_type=jnp.float32)
        tq = tcol_ref[0, pl.ds(lo, GQ), :]
        tkv = tkvh_ref[0, g]
        qb = bcol_ref[0, pl.ds(lo, GQ), :]
        kvb = bkvh_ref[0, g]
        dots = jnp.where(qb != kvb, NEG, dots)
        valid = (tq < seq_len) & (tkv < seq_len)
        dots = jnp.where(~valid, NEG, dots)
        dots = jnp.where(tkv > tq, NEG, dots)
        dots = jnp.where(tq == tkv, -1e-5, dots)
        dots = jnp.where(in_window, dots, NEG)
        m = jnp.max(dots, axis=1, keepdims=True)
        e = jnp.exp(dots - m)
        s = jnp.sum(e, axis=1, keepdims=True)
        lse = jnp.log(s) + m
        bo = lax.dot_general(e, v, (((1,), (0,)), ((), ())),
                             preferred_element_type=jnp.float32) * (1.0 / s)
        so_ref[0, pl.ds(lo, GQ), :] = bo
        lse_ref[0, pl.ds(lo, GQ), :] = lse
        return 0

    lax.fori_loop(0, N_GROUPS, body, 0)


def _halo(x):
    xc = x.reshape(-1, N_GROUPS, GQ)
    xp = jnp.roll(x.reshape(-1, N_CHUNKS, BUCKET_SIZE), 1, axis=1)[:, 0::CPG]
    return jnp.concatenate([xp, xc], axis=2)[:, :, None, :]


def _attention(sqk, sv, st, sbkt, seq_len):
    g = N * NUM_HEADS
    tcol = st[..., None]
    bcol = sbkt[..., None]
    tkvh = _halo(st)
    bkvh = _halo(sbkt)
    seq = jnp.asarray(seq_len, dtype=jnp.int32).reshape(1, 1)
    kernel_fn = pl.pallas_call(
        _attn_kernel,
        grid=(g,),
        in_specs=[
            pl.BlockSpec(memory_space=pltpu.SMEM),
            pl.BlockSpec((1, SEQ_SORT, HEAD_DIM), lambda i: (i, 0, 0)),
            pl.BlockSpec((1, SEQ_SORT, HEAD_DIM), lambda i: (i, 0, 0)),
            pl.BlockSpec((1, SEQ_SORT, 1), lambda i: (i, 0, 0)),
            pl.BlockSpec((1, N_GROUPS, 1, GK), lambda i: (i, 0, 0, 0)),
            pl.BlockSpec((1, SEQ_SORT, 1), lambda i: (i, 0, 0)),
            pl.BlockSpec((1, N_GROUPS, 1, GK), lambda i: (i, 0, 0, 0)),
        ],
        out_specs=[
            pl.BlockSpec((1, SEQ_SORT, HEAD_DIM), lambda i: (i, 0, 0)),
            pl.BlockSpec((1, SEQ_SORT, 1), lambda i: (i, 0, 0)),
        ],
        out_shape=[
            jax.ShapeDtypeStruct((g, SEQ_SORT, HEAD_DIM), jnp.float32),
            jax.ShapeDtypeStruct((g, SEQ_SORT, 1), jnp.float32),
        ],
        interpret=_INTERPRET,
    )
    return kernel_fn(seq, sqk, sv, tcol, tkvh, bcol, bkvh)


def _ln_kernel(x_ref, g_ref, b_ref, out_ref):
    x = x_ref[0]
    mean = jnp.mean(x, axis=1, keepdims=True)
    var = jnp.mean((x - mean) ** 2, axis=1, keepdims=True)
    out_ref[0] = (x - mean) / jnp.sqrt(var + 1e-3) * g_ref[...] + b_ref[...]


def _layernorm(x, gamma, beta):
    bt = 256
    kernel_fn = pl.pallas_call(
        _ln_kernel,
        grid=(N, T // bt),
        in_specs=[
            pl.BlockSpec((1, bt, DIM), lambda n, t: (n, t, 0)),
            pl.BlockSpec((1, DIM), lambda n, t: (0, 0)),
            pl.BlockSpec((1, DIM), lambda n, t: (0, 0)),
        ],
        out_specs=pl.BlockSpec((1, bt, DIM), lambda n, t: (n, t, 0)),
        out_shape=jax.ShapeDtypeStruct((N, T, DIM), jnp.float32),
        interpret=_INTERPRET,
    )
    return kernel_fn(x, gamma.reshape(1, DIM), beta.reshape(1, DIM))


def kernel(inputs, W_Q, b_Q, W_V, b_V, gamma, beta, seq_len):
    rot = _rot_matrix()
    qh, vh, bk0, bk1 = _projection(inputs, W_Q, b_Q, W_V, b_V, rot)

    bkt = jnp.stack([bk0, bk1], axis=1)
    bkt = bkt.transpose(0, 3, 1, 2)
    bkt = bkt.reshape(N * NUM_HEADS, SEQ_SORT)

    qh_flat = qh.reshape(NUM_HEADS * N * T, HEAD_DIM)
    vh_flat = vh.reshape(NUM_HEADS * N * T, HEAD_DIM)
    sqk, sv, st, sbkt, undo = _sc_sort_gather(bkt, qh_flat, vh_flat)

    so, lse = _attention(sqk, sv, st, sbkt, seq_len)

    o_comb = _sc_unsort_combine(
        so.reshape(N * NUM_HEADS * SEQ_SORT, HEAD_DIM), lse[..., 0], undo)
    if isinstance(o_comb, (list, tuple)):
        o_comb = o_comb[0]

    return _layernorm(o_comb.reshape(N, T, DIM), gamma, beta)


# speedup vs baseline: 1.0921x; 1.0565x over previous
import functools

import jax
import jax.numpy as jnp
from jax import lax
from jax.experimental import pallas as pl
from jax.experimental.pallas import tpu as pltpu
from jax.experimental.pallas import tpu_sc as plsc

N = 2
T = 2048
D_MODEL = 2048
DIM = 2048
NUM_HEADS = 16
HEAD_DIM = DIM // NUM_HEADS
NUM_HASHES = 2
BUCKET_SIZE = 64
N_BUCKETS = T // BUCKET_SIZE
N_CHUNKS = NUM_HASHES * N_BUCKETS
SEQ_SORT = NUM_HASHES * T
NEG = -1e9

_INTERPRET = False


def _rot_matrix():
    rkey = jax.random.key(42)
    rots = []
    for h in range(NUM_HEADS):
        kh = jax.random.fold_in(rkey, h)
        r = jax.random.normal(kh, (1, HEAD_DIM, NUM_HASHES, N_BUCKETS // 2),
                              dtype=jnp.float32)
        rots.append(r[0])
    rot = jnp.stack(rots, axis=0)
    return rot.reshape(NUM_HEADS, HEAD_DIM, NUM_HASHES * (N_BUCKETS // 2))


def _proj_kernel(x_ref, wq_ref, bq_ref, wv_ref, bv_ref, rot_ref,
                 qh_ref, vh_ref, bk0_ref, bk1_ref):
    x = x_ref[0]
    q = jnp.dot(x, wq_ref[...], preferred_element_type=jnp.float32) + bq_ref[...]
    v = jnp.dot(x, wv_ref[...], preferred_element_type=jnp.float32) + bv_ref[...]
    bt = q.shape[0]
    for h in range(NUM_HEADS):
        qh = q[:, h * HEAD_DIM:(h + 1) * HEAD_DIM]
        qh_ref[h, 0] = qh
        vh_ref[h, 0] = v[:, h * HEAD_DIM:(h + 1) * HEAD_DIM]
        rh = rot_ref[h]
        hh = jnp.dot(qh, rh, preferred_element_type=jnp.float32)
        for hsh, out_ref in ((0, bk0_ref), (1, bk1_ref)):
            r = hh[:, hsh * (N_BUCKETS // 2):(hsh + 1) * (N_BUCKETS // 2)]
            full = jnp.concatenate([r, -r], axis=1)
            m = jnp.max(full, axis=1, keepdims=True)
            iota = lax.broadcasted_iota(jnp.int32, (bt, N_BUCKETS), 1)
            idx = jnp.min(jnp.where(full == m, iota, N_BUCKETS), axis=1,
                          keepdims=True)
            out_ref[0, :, h:h + 1] = idx + hsh * N_BUCKETS


def _projection(x, w_q, b_q, w_v, b_v, rot):
    bt = 256
    grid = (N, T // bt)
    kernel_fn = pl.pallas_call(
        _proj_kernel,
        grid=grid,
        in_specs=[
            pl.BlockSpec((1, bt, D_MODEL), lambda n, t: (n, t, 0)),
            pl.BlockSpec((D_MODEL, DIM), lambda n, t: (0, 0)),
            pl.BlockSpec((1, DIM), lambda n, t: (0, 0)),
            pl.BlockSpec((D_MODEL, DIM), lambda n, t: (0, 0)),
            pl.BlockSpec((1, DIM), lambda n, t: (0, 0)),
            pl.BlockSpec((NUM_HEADS, HEAD_DIM, N_BUCKETS), lambda n, t: (0, 0, 0)),
        ],
        out_specs=[
            pl.BlockSpec((NUM_HEADS, 1, bt, HEAD_DIM), lambda n, t: (0, n, t, 0)),
            pl.BlockSpec((NUM_HEADS, 1, bt, HEAD_DIM), lambda n, t: (0, n, t, 0)),
            pl.BlockSpec((1, bt, NUM_HEADS), lambda n, t: (n, t, 0)),
            pl.BlockSpec((1, bt, NUM_HEADS), lambda n, t: (n, t, 0)),
        ],
        out_shape=[
            jax.ShapeDtypeStruct((NUM_HEADS, N, T, HEAD_DIM), jnp.float32),
            jax.ShapeDtypeStruct((NUM_HEADS, N, T, HEAD_DIM), jnp.float32),
            jax.ShapeDtypeStruct((N, T, NUM_HEADS), jnp.int32),
            jax.ShapeDtypeStruct((N, T, NUM_HEADS), jnp.int32),
        ],
        interpret=_INTERPRET,
    )
    return kernel_fn(x, w_q, b_q.reshape(1, DIM), w_v, b_v.reshape(1, DIM), rot)


CHW = 128
N_GCHUNK = SEQ_SORT // CHW


def _sc_sort_gather(bkt, qh_flat, vh_flat):
    G = N * NUM_HEADS
    mesh = plsc.VectorSubcoreMesh(core_axis_name="c", subcore_axis_name="s")

    @functools.partial(
        pl.kernel, mesh=mesh,
        compiler_params=pltpu.CompilerParams(needs_layout_passes=False),
        out_type=[
            jax.ShapeDtypeStruct((G, SEQ_SORT, HEAD_DIM), jnp.float32),
            jax.ShapeDtypeStruct((G, SEQ_SORT, HEAD_DIM), jnp.float32),
            jax.ShapeDtypeStruct((G, SEQ_SORT), jnp.int32),
            jax.ShapeDtypeStruct((G, SEQ_SORT), jnp.int32),
            jax.ShapeDtypeStruct((G, SEQ_SORT), jnp.int32),
        ],
        scratch_types=[
            pltpu.VMEM((SEQ_SORT,), jnp.int32),
            pltpu.VMEM((SEQ_SORT + 16,), jnp.int32),
            pltpu.VMEM((SEQ_SORT + 16,), jnp.int32),
            pltpu.VMEM((16,), jnp.int32),
            pltpu.VMEM((SEQ_SORT + 16,), jnp.int32),
            pltpu.VMEM((SEQ_SORT,), jnp.int32),
            pltpu.VMEM((SEQ_SORT,), jnp.int32),
            pltpu.VMEM((SEQ_SORT,), jnp.int32),
            pltpu.VMEM((CHW,), jnp.int32),
            pltpu.VMEM((CHW,), jnp.int32),
            pltpu.VMEM((SEQ_SORT,), jnp.int32),
            pltpu.VMEM((CHW, HEAD_DIM), jnp.float32),
            pltpu.VMEM((CHW, HEAD_DIM), jnp.float32),
            pltpu.VMEM((CHW, HEAD_DIM), jnp.float32),
            pltpu.VMEM((CHW, HEAD_DIM), jnp.float32),
            pltpu.SemaphoreType.DMA,
            pltpu.SemaphoreType.DMA,
            pltpu.SemaphoreType.DMA,
            pltpu.SemaphoreType.DMA,
        ],
    )
    def k(bkt_hbm, qh_hbm, vh_hbm,
          sqk_hbm, sv_hbm, st_hbm, sbkt_hbm, undo_hbm,
          bkt_v, bins_b_v, bins_i_v, starts_v, ids_v, st_v, sbkt_v, qidx_v,
          idx_a, idx_b, undo_v, qr_a, qr_b, vr_a, vr_b,
          sq_a, sq_b, sv_a, sv_b):
        cid = lax.axis_index("c")
        sid = lax.axis_index("s")
        w = sid * 2 + cid
        n = w // NUM_HEADS
        h = lax.rem(w, NUM_HEADS)
        base = (h * N + n) * T

        pltpu.sync_copy(bkt_hbm.at[w], bkt_v)
        lane = lax.iota(jnp.int32, 16)

        starts_v[pl.ds(0, 16)] = jnp.zeros((16,), jnp.int32) + SEQ_SORT
        zeros16 = jnp.zeros((16,), jnp.int32)

        def lvl1(bin_lo, glo, ghi, off0):
            def outer(bi, off):
                binv = bin_lo + bi
                plsc.store_scatter(starts_v, [zeros16 + binv], zeros16 + off,
                                   mask=lane == 0)

                def inner(g, off2):
                    vec = bkt_v[pl.ds(g * 16, 16)]
                    mask = lax.shift_right_logical(vec, 3) == binv
                    csum = plsc.cumsum(mask.astype(jnp.int32))
                    tgt = off2 + csum - 1
                    plsc.store_scatter(bins_b_v, [tgt], vec, mask=mask)
                    plsc.store_scatter(bins_i_v, [tgt], g * 16 + lane,
                                       mask=mask)
                    return off2 + csum[15]
                return lax.fori_loop(glo, ghi, inner, off)
            return lax.fori_loop(0, 4, outer, off0)

        off = lvl1(0, 0, T // 16, 0)
        lvl1(4, T // 16, SEQ_SORT // 16, off)

        def lvl2(v, off2):
            binv = lax.shift_right_logical(v, 3)
            s0 = plsc.load_gather(starts_v, [zeros16 + binv])[0]
            s1 = plsc.load_gather(starts_v, [zeros16 + binv + 1])[0]
            g0 = s0 // 16
            g1 = (s1 + 15) // 16

            def inner(g, o):
                vec = bins_b_v[pl.ds(g * 16, 16)]
                mask = vec == v
                idxs = bins_i_v[pl.ds(g * 16, 16)]
                csum = plsc.cumsum(mask.astype(jnp.int32))
                plsc.store_scatter(ids_v, [o + csum - 1], idxs, mask=mask)
                return o + csum[15]
            return lax.fori_loop(g0, g1, inner, off2)
        lax.fori_loop(0, N_CHUNKS, lvl2, 0)

        def derive(g, _):
            idv = ids_v[pl.ds(g * 16, 16)]
            sb = plsc.load_gather(bkt_v, [idv])
            t = lax.rem(idv, T)
            st_v[pl.ds(g * 16, 16)] = t
            sbkt_v[pl.ds(g * 16, 16)] = sb
            qidx_v[pl.ds(g * 16, 16)] = base + t
            plsc.store_scatter(undo_v, [idv], g * 16 + lane)
            return 0
        lax.fori_loop(0, SEQ_SORT // 16, derive, 0)

        pltpu.sync_copy(st_v, st_hbm.at[w])
        pltpu.sync_copy(sbkt_v, sbkt_hbm.at[w])
        pltpu.sync_copy(undo_v, undo_hbm.at[w])

        ibufs = (idx_a, idx_b)
        qrows = (qr_a, qr_b)
        vrows = (vr_a, vr_b)
        qsems = (sq_a, sq_b)
        vsems = (sv_a, sv_b)

        def fill_idx(c, dst):
            for i in range(CHW // 16):
                dst[pl.ds(i * 16, 16)] = qidx_v[pl.ds(c * CHW + i * 16, 16)]

        fill_idx(0, ibufs[0])
        pltpu.async_copy(qh_hbm.at[ibufs[0]], qrows[0], qsems[0])
        pltpu.async_copy(vh_hbm.at[ibufs[0]], vrows[0], vsems[0])

        def gather(i, _):
            for b in (0, 1):
                c = 2 * i + b
                nb = 1 - b

                @pl.when(c + 1 < N_GCHUNK)
                def _():
                    fill_idx(c + 1, ibufs[nb])
                    pltpu.async_copy(qh_hbm.at[ibufs[nb]], qrows[nb],
                                     qsems[nb])
                    pltpu.async_copy(vh_hbm.at[ibufs[nb]], vrows[nb],
                                     vsems[nb])
                pltpu.make_async_copy(qh_hbm.at[ibufs[b]], qrows[b],
                                      qsems[b]).wait()
                pltpu.sync_copy(qrows[b], sqk_hbm.at[w, pl.ds(c * CHW, CHW), :])
                pltpu.make_async_copy(vh_hbm.at[ibufs[b]], vrows[b],
                                      vsems[b]).wait()
                pltpu.sync_copy(vrows[b], sv_hbm.at[w, pl.ds(c * CHW, CHW), :])
            return 0
        lax.fori_loop(0, N_GCHUNK // 2, gather, 0)

    return k(bkt, qh_flat, vh_flat)


TCH = 128
N_TCHUNK = T // TCH


def _sc_unsort_combine(so_flat, lse, undo):
    G = N * NUM_HEADS
    mesh = plsc.VectorSubcoreMesh(core_axis_name="c", subcore_axis_name="s")

    @functools.partial(
        pl.kernel, mesh=mesh,
        compiler_params=pltpu.CompilerParams(needs_layout_passes=False),
        out_type=[
            jax.ShapeDtypeStruct((N * T, DIM), jnp.float32),
        ],
        scratch_types=[
            pltpu.VMEM((SEQ_SORT,), jnp.float32),
            pltpu.VMEM((SEQ_SORT,), jnp.int32),
            pltpu.VMEM((T,), jnp.float32),
            pltpu.VMEM((T,), jnp.float32),
            pltpu.VMEM((TCH,), jnp.int32),
            pltpu.VMEM((TCH,), jnp.int32),
            pltpu.VMEM((TCH,), jnp.int32),
            pltpu.VMEM((TCH,), jnp.int32),
            pltpu.VMEM((TCH, HEAD_DIM), jnp.float32),
            pltpu.VMEM((TCH, HEAD_DIM), jnp.float32),
            pltpu.VMEM((TCH, HEAD_DIM), jnp.float32),
            pltpu.VMEM((TCH, HEAD_DIM), jnp.float32),
            pltpu.SemaphoreType.DMA,
            pltpu.SemaphoreType.DMA,
            pltpu.SemaphoreType.DMA,
            pltpu.SemaphoreType.DMA,
        ],
    )
    def k(so_hbm, lse_hbm, undo_hbm, o_hbm,
          lse_v, undo_v, w0_v, w1_v, i0_a, i0_b, i1_a, i1_b,
          r0_a, r0_b, r1_a, r1_b, s0_a, s0_b, s1_a, s1_b):
        cid = lax.axis_index("c")
        sid = lax.axis_index("s")
        w = sid * 2 + cid
        n = w // NUM_HEADS
        h = lax.rem(w, NUM_HEADS)
        srow = w * SEQ_SORT

        pltpu.sync_copy(lse_hbm.at[w], lse_v)
        pltpu.sync_copy(undo_hbm.at[w], undo_v)

        def weights(g, _):
            p0 = undo_v[pl.ds(g * 16, 16)]
            p1 = undo_v[pl.ds(T + g * 16, 16)]
            l0 = plsc.load_gather(lse_v, [p0])
            l1 = plsc.load_gather(lse_v, [p1])
            m = jnp.maximum(l0, l1)
            e0 = jnp.exp(l0 - m)
            e1 = jnp.exp(l1 - m)
            s = e0 + e1
            w0_v[pl.ds(g * 16, 16)] = e0 / s
            w1_v[pl.ds(g * 16, 16)] = e1 / s
            return 0
        lax.fori_loop(0, T // 16, weights, 0)

        i0 = (i0_a, i0_b)
        i1 = (i1_a, i1_b)
        r0 = (r0_a, r0_b)
        r1 = (r1_a, r1_b)
        s0 = (s0_a, s0_b)
        s1 = (s1_a, s1_b)

        def fill_idx(c, d0, d1):
            for i in range(TCH // 16):
                d0[pl.ds(i * 16, 16)] = srow + undo_v[
                    pl.ds(c * TCH + i * 16, 16)]
                d1[pl.ds(i * 16, 16)] = srow + undo_v[
                    pl.ds(T + c * TCH + i * 16, 16)]

        fill_idx(0, i0[0], i1[0])
        pltpu.async_copy(so_hbm.at[i0[0]], r0[0], s0[0])
        pltpu.async_copy(so_hbm.at[i1[0]], r1[0], s1[0])

        def body(i, _):
            for b in (0, 1):
                c = 2 * i + b
                nb = 1 - b

                @pl.when(c + 1 < N_TCHUNK)
                def _():
                    fill_idx(c + 1, i0[nb], i1[nb])
                    pltpu.async_copy(so_hbm.at[i0[nb]], r0[nb], s0[nb])
                    pltpu.async_copy(so_hbm.at[i1[nb]], r1[nb], s1[nb])
                pltpu.make_async_copy(so_hbm.at[i0[b]], r0[b], s0[b]).wait()
                pltpu.make_async_copy(so_hbm.at[i1[b]], r1[b], s1[b]).wait()

                def comb(rg, _):
                    wv0 = w0_v[pl.ds(c * TCH + rg * 16, 16)]
                    wv1 = w1_v[pl.ds(c * TCH + rg * 16, 16)]
                    for rr in range(16):
                        a = wv0[rr]
                        bb = wv1[rr]
                        r = rg * 16 + rr
                        for u in range(HEAD_DIM // 16):
                            r0[b][r, pl.ds(u * 16, 16)] = (
                                a * r0[b][r, pl.ds(u * 16, 16)]
                                + bb * r1[b][r, pl.ds(u * 16, 16)])
                    return 0
                lax.fori_loop(0, TCH // 16, comb, 0)
                pltpu.sync_copy(
                    r0[b],
                    o_hbm.at[pl.ds(n * T + c * TCH, TCH),
                             pl.ds(h * HEAD_DIM, HEAD_DIM)])
            return 0
        lax.fori_loop(0, N_TCHUNK // 2, body, 0)

    return k(so_flat, lse, undo)


CPG = 8
GQ = CPG * BUCKET_SIZE
GK = GQ + BUCKET_SIZE
N_GROUPS = N_CHUNKS // CPG


def _attn_kernel(seq_ref, sqk_ref, sv_ref, tcol_ref, tkvh_ref, bcol_ref,
                 bkvh_ref, so_ref, lse_ref):
    seq_len = seq_ref[0, 0]
    scale = float(HEAD_DIM) ** -0.5
    cs = BUCKET_SIZE

    rc = lax.broadcasted_iota(jnp.int32, (GQ, GK), 0) // cs
    jc = lax.broadcasted_iota(jnp.int32, (GQ, GK), 1) // cs
    in_window = (jc == rc) | (jc == rc + 1)

    def body(g, _):
        lo = g * GQ
        po = (lax.rem(g * CPG + N_CHUNKS - 1, N_CHUNKS)) * cs
        q = sqk_ref[0, pl.ds(lo, GQ), :]
        kp = sqk_ref[0, pl.ds(po, cs), :]
        k = jnp.concatenate([kp, q], axis=0)
        vp = sv_ref[0, pl.ds(po, cs), :]
        v = jnp.concatenate([vp, sv_ref[0, pl.ds(lo, GQ), :]], axis=0)
        knorm = k / (jnp.sqrt(jnp.sum(k * k, axis=1, keepdims=True)) + 1e-6)
        dots = lax.dot_general(q * scale, knorm, (((1,), (1,)), ((), ())),
                               preferred_element_type=jnp.float32)
        tq = tcol_ref[0, pl.ds(lo, GQ), :]
        tkv = tkvh_ref[0, g]
        qb = bcol_ref[0, pl.ds(lo, GQ), :]
        kvb = bkvh_ref[0, g]
        dots = jnp.where(qb != kvb, NEG, dots)
        valid = (tq < seq_len) & (tkv < seq_len)
        dots = jnp.where(~valid, NEG, dots)
        dots = jnp.where(tkv > tq, NEG, dots)
        dots = jnp.where(tq == tkv, -1e-5, dots)
        dots = jnp.where(in_window, dots, NEG)
        m = jnp.max(dots, axis=1, keepdims=True)
        e = jnp.exp(dots - m)
        s = jnp.sum(e, axis=1, keepdims=True)
        lse = jnp.log(s) + m
        bo = lax.dot_general(e, v, (((1,), (0,)), ((), ())),
                             preferred_element_type=jnp.float32) * (1.0 / s)
        so_ref[0, pl.ds(lo, GQ), :] = bo
        lse_ref[0, pl.ds(lo, GQ), :] = lse
        return 0

    lax.fori_loop(0, N_GROUPS, body, 0)


def _halo(x):
    xc = x.reshape(-1, N_GROUPS, GQ)
    xp = jnp.roll(x.reshape(-1, N_CHUNKS, BUCKET_SIZE), 1, axis=1)[:, 0::CPG]
    return jnp.concatenate([xp, xc], axis=2)[:, :, None, :]


def _attention(sqk, sv, st, sbkt, seq_len):
    g = N * NUM_HEADS
    tcol = st[..., None]
    bcol = sbkt[..., None]
    tkvh = _halo(st)
    bkvh = _halo(sbkt)
    seq = jnp.asarray(seq_len, dtype=jnp.int32).reshape(1, 1)
    kernel_fn = pl.pallas_call(
        _attn_kernel,
        grid=(g,),
        in_specs=[
            pl.BlockSpec(memory_space=pltpu.SMEM),
            pl.BlockSpec((1, SEQ_SORT, HEAD_DIM), lambda i: (i, 0, 0)),
            pl.BlockSpec((1, SEQ_SORT, HEAD_DIM), lambda i: (i, 0, 0)),
            pl.BlockSpec((1, SEQ_SORT, 1), lambda i: (i, 0, 0)),
            pl.BlockSpec((1, N_GROUPS, 1, GK), lambda i: (i, 0, 0, 0)),
            pl.BlockSpec((1, SEQ_SORT, 1), lambda i: (i, 0, 0)),
            pl.BlockSpec((1, N_GROUPS, 1, GK), lambda i: (i, 0, 0, 0)),
        ],
        out_specs=[
            pl.BlockSpec((1, SEQ_SORT, HEAD_DIM), lambda i: (i, 0, 0)),
            pl.BlockSpec((1, SEQ_SORT, 1), lambda i: (i, 0, 0)),
        ],
        out_shape=[
            jax.ShapeDtypeStruct((g, SEQ_SORT, HEAD_DIM), jnp.float32),
            jax.ShapeDtypeStruct((g, SEQ_SORT, 1), jnp.float32),
        ],
        interpret=_INTERPRET,
    )
    return kernel_fn(seq, sqk, sv, tcol, tkvh, bcol, bkvh)


def _ln_kernel(x_ref, g_ref, b_ref, out_ref):
    x = x_ref[0]
    mean = jnp.mean(x, axis=1, keepdims=True)
    var = jnp.mean((x - mean) ** 2, axis=1, keepdims=True)
    out_ref[0] = (x - mean) / jnp.sqrt(var + 1e-3) * g_ref[...] + b_ref[...]


def _layernorm(x, gamma, beta):
    bt = 256
    kernel_fn = pl.pallas_call(
        _ln_kernel,
        grid=(N, T // bt),
        in_specs=[
            pl.BlockSpec((1, bt, DIM), lambda n, t: (n, t, 0)),
            pl.BlockSpec((1, DIM), lambda n, t: (0, 0)),
            pl.BlockSpec((1, DIM), lambda n, t: (0, 0)),
        ],
        out_specs=pl.BlockSpec((1, bt, DIM), lambda n, t: (n, t, 0)),
        out_shape=jax.ShapeDtypeStruct((N, T, DIM), jnp.float32),
        interpret=_INTERPRET,
    )
    return kernel_fn(x, gamma.reshape(1, DIM), beta.reshape(1, DIM))


def kernel(inputs, W_Q, b_Q, W_V, b_V, gamma, beta, seq_len):
    rot = _rot_matrix()
    qh, vh, bk0, bk1 = _projection(inputs, W_Q, b_Q, W_V, b_V, rot)

    bkt = jnp.stack([bk0, bk1], axis=1)
    bkt = bkt.transpose(0, 3, 1, 2)
    bkt = bkt.reshape(N * NUM_HEADS, SEQ_SORT)

    qh_flat = qh.reshape(NUM_HEADS * N * T, HEAD_DIM)
    vh_flat = vh.reshape(NUM_HEADS * N * T, HEAD_DIM)
    sqk, sv, st, sbkt, undo = _sc_sort_gather(bkt, qh_flat, vh_flat)

    so, lse = _attention(sqk, sv, st, sbkt, seq_len)

    o_comb = _sc_unsort_combine(
        so.reshape(N * NUM_HEADS * SEQ_SORT, HEAD_DIM), lse[..., 0], undo)
    if isinstance(o_comb, (list, tuple)):
        o_comb = o_comb[0]

    return _layernorm(o_comb.reshape(N, T, DIM), gamma, beta)
